# baseline (speedup 1.0000x reference)
import jax
import jax.numpy as jnp
from jax.experimental import pallas as pl
from jax.experimental.pallas import tpu as pltpu

B, N, D = 32, 576, 768
ROWS = B * N
BLK = 1536


def _body(idx_ref, embed_ref, tok_ref, out_ref):
    row = embed_ref[pl.ds(idx_ref[0], 1), :]
    out_ref[...] = tok_ref[...] + row


def kernel(tokens, embed, frame_idx):
    idx = jnp.asarray(frame_idx, dtype=jnp.int32).reshape((1,))
    tok2 = tokens.reshape(ROWS, D)
    out = pl.pallas_call(
        _body,
        grid=(ROWS // BLK,),
        in_specs=[
            pl.BlockSpec(memory_space=pltpu.SMEM),
            pl.BlockSpec((embed.shape[0], D), lambda i: (0, 0)),
            pl.BlockSpec((BLK, D), lambda i: (i, 0)),
        ],
        out_specs=pl.BlockSpec((BLK, D), lambda i: (i, 0)),
        out_shape=jax.ShapeDtypeStruct((ROWS, D), tokens.dtype),
    )(idx, embed, tok2)
    return out.reshape(B, N, D)


# speedup vs baseline: 1.0266x; 1.0266x over previous
import jax
import jax.numpy as jnp
from jax.experimental import pallas as pl
from jax.experimental.pallas import tpu as pltpu

B, N, D = 32, 576, 768
ROWS = B * N
BLK = 2304


def _body(idx_ref, embed_ref, tok_ref, out_ref):
    row = embed_ref[pl.ds(idx_ref[0], 1), :]
    out_ref[...] = tok_ref[...] + row


def kernel(tokens, embed, frame_idx):
    idx = jnp.asarray(frame_idx, dtype=jnp.int32).reshape((1,))
    tok2 = tokens.reshape(ROWS, D)
    out = pl.pallas_call(
        _body,
        grid=(ROWS // BLK,),
        in_specs=[
            pl.BlockSpec(memory_space=pltpu.SMEM),
            pl.BlockSpec((embed.shape[0], D), lambda i: (0, 0)),
            pl.BlockSpec((BLK, D), lambda i: (i, 0)),
        ],
        out_specs=pl.BlockSpec((BLK, D), lambda i: (i, 0)),
        out_shape=jax.ShapeDtypeStruct((ROWS, D), tokens.dtype),
    )(idx, embed, tok2)
    return out.reshape(B, N, D)


# speedup vs baseline: 1.0419x; 1.0148x over previous
import jax
import jax.numpy as jnp
from jax.experimental import pallas as pl
from jax.experimental.pallas import tpu as pltpu

B, N, D = 32, 576, 768
ROWS = B * N
BLK = 3072


def _body(idx_ref, embed_ref, tok_ref, out_ref):
    row = embed_ref[pl.ds(idx_ref[0], 1), :]
    out_ref[...] = tok_ref[...] + row


def kernel(tokens, embed, frame_idx):
    idx = jnp.asarray(frame_idx, dtype=jnp.int32).reshape((1,))
    tok2 = tokens.reshape(ROWS, D)
    out = pl.pallas_call(
        _body,
        grid=(ROWS // BLK,),
        in_specs=[
            pl.BlockSpec(memory_space=pltpu.SMEM),
            pl.BlockSpec((embed.shape[0], D), lambda i: (0, 0)),
            pl.BlockSpec((BLK, D), lambda i: (i, 0)),
        ],
        out_specs=pl.BlockSpec((BLK, D), lambda i: (i, 0)),
        out_shape=jax.ShapeDtypeStruct((ROWS, D), tokens.dtype),
    )(idx, embed, tok2)
    return out.reshape(B, N, D)


# speedup vs baseline: 1.0559x; 1.0134x over previous
import jax
import jax.numpy as jnp
from jax.experimental import pallas as pl
from jax.experimental.pallas import tpu as pltpu

B, N, D = 32, 576, 768
ROWS = B * N
BLK = 4608


def _body(idx_ref, embed_ref, tok_ref, out_ref):
    row = embed_ref[pl.ds(idx_ref[0], 1), :]
    out_ref[...] = tok_ref[...] + row


def kernel(tokens, embed, frame_idx):
    idx = jnp.asarray(frame_idx, dtype=jnp.int32).reshape((1,))
    tok2 = tokens.reshape(ROWS, D)
    out = pl.pallas_call(
        _body,
        grid=(ROWS // BLK,),
        in_specs=[
            pl.BlockSpec(memory_space=pltpu.SMEM),
            pl.BlockSpec((embed.shape[0], D), lambda i: (0, 0)),
            pl.BlockSpec((BLK, D), lambda i: (i, 0)),
        ],
        out_specs=pl.BlockSpec((BLK, D), lambda i: (i, 0)),
        out_shape=jax.ShapeDtypeStruct((ROWS, D), tokens.dtype),
    )(idx, embed, tok2)
    return out.reshape(B, N, D)
